# baseline (speedup 1.0000x reference)
import jax
import jax.numpy as jnp
from jax import lax
from jax.experimental import pallas as pl
from jax.experimental.pallas import tpu as pltpu, tpu_sc as plsc

_B, _S, _D = 4, 4096, 1024
_Q = 1024
_QS0 = _S - _Q
_NC, _NS = 2, 16
_NW = _NC * _NS
_SW = _Q // _NW
_R = 8
_CH = _SW // _R
_NSL = 3
_BS = 512


def _tc_add(x_ref, p_ref, o_ref):
    o_ref[...] = x_ref[...] + p_ref[...]


def _sc_body(x_hbm, pos_hbm, out_hbm,
             x00, x01, x02, x03, x10, x11, x12, x13, x20, x21, x22, x23,
             pb0, pb1, pb2,
             ls0, ls1, ls2, ss0, ss1, ss2, ps0, ps1, ps2):
    xb = [[x00, x01, x02, x03], [x10, x11, x12, x13], [x20, x21, x22, x23]]
    pb, ls, ss, ps = [pb0, pb1, pb2], [ls0, ls1, ls2], [ss0, ss1, ss2], [ps0, ps1, ps2]
    wid = lax.axis_index("s") * _NC + lax.axis_index("c")
    s0 = _QS0 + wid * _SW
    o0 = wid * _SW

    def load_chunk(c):
        sl = c % _NSL
        row = s0 + c * _R
        pltpu.async_copy(pos_hbm.at[pl.ds(row, _R), :], pb[sl], ps[sl])
        for b in range(_B):
            pltpu.async_copy(x_hbm.at[b, pl.ds(row, _R), :], xb[sl][b], ls[sl])

    def wait_chunk(c):
        sl = c % _NSL
        row = s0 + c * _R
        pltpu.make_async_copy(
            pos_hbm.at[pl.ds(row, _R), :], pb[sl], ps[sl]).wait()
        for b in range(_B):
            pltpu.make_async_copy(
                x_hbm.at[b, pl.ds(row, _R), :], xb[sl][b], ls[sl]).wait()

    load_chunk(0)
    load_chunk(1)
    pending_stores = [None] * _NSL

    for c in range(_CH):
        sl = c % _NSL
        orow = o0 + c * _R
        if c + 2 < _CH:
            nsl = (c + 2) % _NSL
            if pending_stores[nsl] is not None:
                for h in pending_stores[nsl]:
                    h.wait()
                pending_stores[nsl] = None
            load_chunk(c + 2)
        wait_chunk(c)

        pc, xc = pb[sl], xb[sl]

        @plsc.parallel_loop(0, _D, step=16)
        def _add(i):
            for r in range(_R):
                v = pc[r, pl.ds(i, 16)]
                for b in range(_B):
                    plsc.addupdate(xc[b].at[r, pl.ds(i, 16)], v)

        pending_stores[sl] = [
            pltpu.async_copy(xc[b], out_hbm.at[b, pl.ds(orow, _R), :], ss[sl])
            for b in range(_B)
        ]

    for hs in pending_stores:
        if hs is not None:
            for h in hs:
                h.wait()


def kernel(x, pos_table):
    B, S, D = x.shape

    sc_run = pl.kernel(
        _sc_body,
        out_type=jax.ShapeDtypeStruct((B, _Q, D), jnp.float32),
        mesh=plsc.VectorSubcoreMesh(core_axis_name="c", subcore_axis_name="s"),
        scratch_types=(
            [pltpu.VMEM((_R, _D), jnp.float32)] * 15
            + [pltpu.SemaphoreType.DMA] * 9
        ),
        compiler_params=pltpu.CompilerParams(use_tc_tiling_on_sc=True),
    )
    sc_out = sc_run(x, pos_table)

    tc_out = pl.pallas_call(
        _tc_add,
        grid=(_QS0 // _BS, B),
        in_specs=[
            pl.BlockSpec((None, _BS, D), lambda s, b: (b, s, 0)),
            pl.BlockSpec((_BS, D), lambda s, b: (s, 0)),
        ],
        out_specs=pl.BlockSpec((None, _BS, D), lambda s, b: (b, s, 0)),
        out_shape=jax.ShapeDtypeStruct((B, S, D), x.dtype),
    )(x, pos_table)

    return lax.dynamic_update_slice(tc_out, sc_out, (0, _QS0, 0))


# speedup vs baseline: 1.0109x; 1.0109x over previous
import jax
import jax.numpy as jnp
from jax import lax
from jax.experimental import pallas as pl
from jax.experimental.pallas import tpu as pltpu, tpu_sc as plsc

_B, _S, _D = 4, 4096, 1024
_NC, _NS = 2, 16
_NW = _NC * _NS
_SW = _S // _NW
_R = 8
_CH = _SW // _R
_NSL = 3


def _sc_body(x_hbm, pos_hbm, out_hbm,
             x00, x01, x02, x03, x10, x11, x12, x13, x20, x21, x22, x23,
             pb0, pb1, pb2,
             ls0, ls1, ls2, ss0, ss1, ss2, ps0, ps1, ps2):
    xb = [[x00, x01, x02, x03], [x10, x11, x12, x13], [x20, x21, x22, x23]]
    pb, ls, ss, ps = [pb0, pb1, pb2], [ls0, ls1, ls2], [ss0, ss1, ss2], [ps0, ps1, ps2]
    wid = lax.axis_index("s") * _NC + lax.axis_index("c")
    s0 = wid * _SW

    def load_chunk(c):
        sl = c % _NSL
        row = s0 + c * _R
        pltpu.async_copy(pos_hbm.at[pl.ds(row, _R), :], pb[sl], ps[sl])
        for b in range(_B):
            pltpu.async_copy(x_hbm.at[b, pl.ds(row, _R), :], xb[sl][b], ls[sl])

    def wait_chunk(c):
        sl = c % _NSL
        row = s0 + c * _R
        pltpu.make_async_copy(
            pos_hbm.at[pl.ds(row, _R), :], pb[sl], ps[sl]).wait()
        for b in range(_B):
            pltpu.make_async_copy(
                x_hbm.at[b, pl.ds(row, _R), :], xb[sl][b], ls[sl]).wait()

    load_chunk(0)
    load_chunk(1)
    pending_stores = [None] * _NSL

    for c in range(_CH):
        sl = c % _NSL
        row = s0 + c * _R
        if c + 2 < _CH:
            nsl = (c + 2) % _NSL
            if pending_stores[nsl] is not None:
                for h in pending_stores[nsl]:
                    h.wait()
                pending_stores[nsl] = None
            load_chunk(c + 2)
        wait_chunk(c)

        pc, xc = pb[sl], xb[sl]

        @plsc.parallel_loop(0, _D, step=16)
        def _add(i):
            for r in range(_R):
                v = pc[r, pl.ds(i, 16)]
                for b in range(_B):
                    plsc.addupdate(xc[b].at[r, pl.ds(i, 16)], v)

        pending_stores[sl] = [
            pltpu.async_copy(xc[b], out_hbm.at[b, pl.ds(row, _R), :], ss[sl])
            for b in range(_B)
        ]

    for hs in pending_stores:
        if hs is not None:
            for h in hs:
                h.wait()


def kernel(x, pos_table):
    B, S, D = x.shape
    run = pl.kernel(
        _sc_body,
        out_type=jax.ShapeDtypeStruct((B, S, D), jnp.float32),
        mesh=plsc.VectorSubcoreMesh(core_axis_name="c", subcore_axis_name="s"),
        scratch_types=(
            [pltpu.VMEM((_R, _D), jnp.float32)] * 15
            + [pltpu.SemaphoreType.DMA] * 9
        ),
        compiler_params=pltpu.CompilerParams(use_tc_tiling_on_sc=True),
    )
    return run(x, pos_table)
